# Initial kernel scaffold; baseline (speedup 1.0000x reference)
import jax
import jax.numpy as jnp
from jax.experimental import pallas as pl


def kernel(llr_in, cn_weight, ch_weight, cn_bias, edge_to_vn, edge_to_cn):
    raise NotImplementedError("write your pallas kernel here")


# speedup vs baseline: 1.0113x; 1.0113x over previous
import jax
import jax.numpy as jnp
from jax.experimental import pallas as pl

CLIP = 20.0
ITERS = 10
N = 26112
M = 17664
E = 121344


def _final_scale(loss_sum_ref, out_ref):
    out_ref[...] = loss_sum_ref[...] / ITERS


def kernel(llr_in, cn_weight, ch_weight, cn_bias, edge_to_vn, edge_to_cn):
    c2v = jnp.zeros(llr_in.shape[:1] + edge_to_vn.shape, dtype=llr_in.dtype)
    sum_llr = jnp.zeros_like(llr_in)
    loss = 0.0
    for it in range(ITERS):
        w_ch = llr_in * ch_weight[it]
        v2c = jnp.take(w_ch + sum_llr, edge_to_vn, axis=1) - c2v
        v2c = jnp.clip(v2c, -CLIP, CLIP)
        a = jnp.abs(v2c).T
        sgn = jnp.where(v2c >= 0, 1.0, -1.0).T
        neg = (v2c < 0).astype(jnp.int32).T
        min1 = jax.ops.segment_min(a, edge_to_cn, num_segments=M)
        min1_g = jnp.take(min1, edge_to_cn, axis=0)
        is_min = a <= min1_g
        a2 = jnp.where(is_min, jnp.inf, a)
        min2 = jax.ops.segment_min(a2, edge_to_cn, num_segments=M)
        min2 = jnp.where(jnp.isinf(min2), min1, min2)
        min2_g = jnp.take(min2, edge_to_cn, axis=0)
        par = jax.ops.segment_sum(neg, edge_to_cn, num_segments=M) % 2
        sign_tot = 1.0 - 2.0 * jnp.take(par, edge_to_cn, axis=0).astype(a.dtype)
        ext = sign_tot * sgn * jnp.where(is_min, min2_g, min1_g)
        c2v_w = ext.T * cn_weight[it]
        mag = jnp.maximum(jnp.abs(c2v_w) - cn_bias[it], 0.0)
        c2v = jnp.clip(jnp.sign(c2v_w) * mag, -CLIP, CLIP)
        sum_llr = jax.ops.segment_sum(c2v.T, edge_to_vn, num_segments=N).T
        loss = loss + jnp.mean(jax.nn.softplus(-(llr_in + sum_llr)))

    loss_arr = jnp.asarray(loss, jnp.float32).reshape(1, 1)
    out = pl.pallas_call(
        _final_scale,
        out_shape=jax.ShapeDtypeStruct((1, 1), jnp.float32),
    )(loss_arr)
    return out[0, 0]


# speedup vs baseline: 1.1037x; 1.0913x over previous
import jax
import jax.numpy as jnp
from jax import lax
from jax.experimental import pallas as pl
from jax.experimental.pallas import tpu as pltpu
from jax.experimental.pallas import tpu_sc as plsc

N = 26112
M = 17664
E = 121344
B = 128
ITERS = 10
CLIP = 20.0
BIG = 1e9

NC = 2
NS = 16
BC = 16
NCHUNK = B // BC
HALVES = tuple(range(0, BC, 16))
CN_SB = 552
NSB = M // CN_SB
K = 512
ROWS_T = N // NS
RBLKS = [(0, 512), (512, 512), (1024, 512), (1536, 96)]
CAPMAX = E + NSB * (K + 8)


def _sc_body(llr_flat, vn_pad, cn_pad, bounds, regs, wcn, wch, bcn,
             dec_out, c2v_buf,
             wcn_v, wch_v, bcn_v, bounds_v, regs_v, vnb_v, cnb_v,
             rows_v, c2v_v, zero_v, stats1, stats2, statsp,
             idx_acc, sem,
             shared_total, shared_acc):
    c = lax.axis_index("c")
    s = lax.axis_index("s")
    iota = lax.broadcasted_iota(jnp.int32, (16,), 0)

    pltpu.sync_copy(wcn, wcn_v.at[pl.ds(0, 16)])
    pltpu.sync_copy(wch, wch_v.at[pl.ds(0, 16)])
    pltpu.sync_copy(bcn, bcn_v.at[pl.ds(0, 16)])
    pltpu.sync_copy(bounds, bounds_v.at[pl.ds(0, 48)])
    pltpu.sync_copy(regs, regs_v.at[pl.ds(0, 48)])

    def sread(ref, idx):
        return ref[pl.ds(idx, 16)][0]

    def _z(i, _):
        for h in HALVES:
            zero_v[i, pl.ds(h, 16)] = jnp.zeros((16,), jnp.float32)
        return 0
    lax.fori_loop(0, K, _z, 0)

    def edge_pass(pass2, it, sb_abs, cn_lo, e_lo, e_hi):
        a_lo = e_lo - lax.rem(e_lo, 8)
        nb = (e_hi - a_lo + (K - 1)) // K
        ro = pl.multiple_of(c * CAPMAX + sread(regs_v, sb_abs), 8)
        cnw_it = sread(wcn_v, it)
        bia_it = sread(bcn_v, it)

        def block_body(b, carry):
            base = pl.multiple_of(a_lo + b * K, 8)
            start_j = jnp.maximum(e_lo - base, 0)
            end_j = jnp.minimum(e_hi - base, K)
            rb = pl.multiple_of(ro + b * K, 8)
            pltpu.sync_copy(vn_pad.at[pl.ds(base, K)], vnb_v)
            pltpu.sync_copy(cn_pad.at[pl.ds(base, K)], cnb_v.at[pl.ds(0, K)])
            pltpu.async_copy(shared_total.at[vnb_v], rows_v, sem).wait()
            pltpu.sync_copy(c2v_buf.at[pl.ds(rb, K)], c2v_v)

            if not pass2:
                def e1(j, cy):
                    cnp, m1s, m2s, ps = cy
                    cnj = sread(cnb_v, j)
                    ci = cnj - cn_lo
                    rst = cnj != cnp

                    def half(h, m1, m2, p):
                        t = rows_v[j, pl.ds(h, 16)]
                        cc = c2v_v[j, pl.ds(h, 16)]
                        v = jnp.minimum(jnp.maximum(t - cc, -CLIP), CLIP)
                        a = jnp.abs(v)
                        sg = jnp.where(v >= 0, 1.0, -1.0)
                        lt = a < m1
                        m1n = jnp.minimum(m1, a)
                        cand = jnp.where(lt, m1, jnp.where(a > m1, a, BIG))
                        m2n = jnp.minimum(m2, cand)
                        m1f = jnp.where(rst, a, m1n)
                        m2f = jnp.where(rst, jnp.full((16,), BIG), m2n)
                        pf = jnp.where(rst, sg, p * sg)
                        return m1f, m2f, pf

                    new1, new2, newp = [], [], []
                    for hi, h in enumerate(HALVES):
                        m1f, m2f, pf = half(h, m1s[hi], m2s[hi], ps[hi])
                        stats1[ci, pl.ds(h, 16)] = m1f
                        stats2[ci, pl.ds(h, 16)] = m2f
                        statsp[ci, pl.ds(h, 16)] = pf
                        new1.append(m1f)
                        new2.append(m2f)
                        newp.append(pf)
                    return (cnj, tuple(new1), tuple(new2), tuple(newp))

                carry = lax.fori_loop(start_j, end_j, e1, carry)
            else:
                def mk(k16, _):
                    jv = iota + k16 * 16
                    inb = (jv >= start_j) & (jv < end_j)
                    vnk = vnb_v[pl.ds(k16 * 16, 16)]
                    idx_acc[pl.ds(k16 * 16, 16)] = jnp.where(inb, vnk,
                                                             N + iota)
                    return 0
                lax.fori_loop(0, K // 16, mk, 0)

                def e2(j, _):
                    cnj = sread(cnb_v, j)
                    ci = cnj - cn_lo
                    for h in HALVES:
                        t = rows_v[j, pl.ds(h, 16)]
                        cc = c2v_v[j, pl.ds(h, 16)]
                        v = jnp.minimum(jnp.maximum(t - cc, -CLIP), CLIP)
                        a = jnp.abs(v)
                        sg = jnp.where(v >= 0, 1.0, -1.0)
                        m1 = stats1[ci, pl.ds(h, 16)]
                        m2r = stats2[ci, pl.ds(h, 16)]
                        p = statsp[ci, pl.ds(h, 16)]
                        m2 = jnp.where(m2r >= BIG * 0.5, m1, m2r)
                        mag = jnp.where(a <= m1, m2, m1)
                        w = (p * sg * mag) * cnw_it
                        mg = jnp.maximum(jnp.abs(w) - bia_it, 0.0)
                        out = jnp.where(w >= 0, mg, -mg)
                        out = jnp.minimum(jnp.maximum(out, -CLIP), CLIP)
                        c2v_v[j, pl.ds(h, 16)] = out
                    return 0

                lax.fori_loop(start_j, end_j, e2, 0)
                pltpu.sync_copy(c2v_v, c2v_buf.at[pl.ds(rb, K)])
                pltpu.sync_copy(c2v_v, shared_acc.at[idx_acc], add=True)
            return carry

        ones = jnp.ones((16,))
        big = jnp.full((16,), BIG)
        nh = len(HALVES)
        init = (jnp.int32(-1), (big,) * nh, (big,) * nh, (ones,) * nh)
        lax.fori_loop(0, nb, block_body, init)

    def round_body(r, _):
        q = NC * r + c

        chw0 = sread(wch_v, 0)
        for roff, rlen in RBLKS:
            rbase = pl.multiple_of(s * ROWS_T + roff, 8)
            pltpu.sync_copy(
                llr_flat.at[pl.ds(pl.multiple_of(q * N + rbase, 8), rlen)],
                c2v_v.at[pl.ds(0, rlen)])

            def t0(i, _):
                for h in HALVES:
                    ll = c2v_v[i, pl.ds(h, 16)]
                    c2v_v[i, pl.ds(h, 16)] = ll * chw0
                return 0
            lax.fori_loop(0, rlen, t0, 0)
            pltpu.sync_copy(c2v_v.at[pl.ds(0, rlen)],
                            shared_total.at[pl.ds(rbase, rlen)])
            pltpu.sync_copy(zero_v.at[pl.ds(0, rlen)],
                            shared_acc.at[pl.ds(rbase, rlen)])

        @pl.when(s == 0)
        def _():
            pltpu.sync_copy(zero_v.at[pl.ds(0, 16)],
                            shared_acc.at[pl.ds(N, 16)])

        z_lo = sread(regs_v, 2 * s)
        z_hi = sread(regs_v, 2 * s + 2)

        def zc(b, _):
            pltpu.sync_copy(
                zero_v,
                c2v_buf.at[pl.ds(pl.multiple_of(c * CAPMAX + z_lo + b * K, 8),
                                 K)])
            return 0
        lax.fori_loop(0, (z_hi - z_lo) // K, zc, 0)

        plsc.subcore_barrier()

        def iter_body(it, _):
            def sb_body(sb, _):
                sb_abs = 2 * s + sb
                cn_lo = sb_abs * CN_SB
                e_lo = sread(bounds_v, sb_abs)
                e_hi = sread(bounds_v, sb_abs + 1)
                edge_pass(False, it, sb_abs, cn_lo, e_lo, e_hi)
                edge_pass(True, it, sb_abs, cn_lo, e_lo, e_hi)
                return 0
            lax.fori_loop(0, 2, sb_body, 0)
            plsc.subcore_barrier()

            chwn = sread(wch_v, jnp.minimum(it + 1, ITERS - 1))
            for roff, rlen in RBLKS:
                rbase = pl.multiple_of(s * ROWS_T + roff, 8)
                pltpu.sync_copy(shared_acc.at[pl.ds(rbase, rlen)],
                                rows_v.at[pl.ds(0, rlen)])
                pltpu.sync_copy(zero_v.at[pl.ds(0, rlen)],
                                shared_acc.at[pl.ds(rbase, rlen)])
                pltpu.sync_copy(
                    llr_flat.at[pl.ds(pl.multiple_of(q * N + rbase, 8),
                                      rlen)],
                    c2v_v.at[pl.ds(0, rlen)])

                def cr(i, _):
                    for h in HALVES:
                        sm = rows_v[i, pl.ds(h, 16)]
                        ll = c2v_v[i, pl.ds(h, 16)]
                        c2v_v[i, pl.ds(h, 16)] = ll + sm
                        rows_v[i, pl.ds(h, 16)] = ll * chwn + sm
                    return 0
                lax.fori_loop(0, rlen, cr, 0)
                pltpu.sync_copy(c2v_v.at[pl.ds(0, rlen)],
                                dec_out.at[pl.ds(pl.multiple_of(
                                    (it * NCHUNK + q) * N + rbase, 8), rlen)])
                pltpu.sync_copy(rows_v.at[pl.ds(0, rlen)],
                                shared_total.at[pl.ds(rbase, rlen)])
            plsc.subcore_barrier()
            return 0

        lax.fori_loop(0, ITERS, iter_body, 0)
        return 0

    lax.fori_loop(0, NCHUNK // NC, round_body, 0)


def _loss_body(dec_ref, out_ref):
    @pl.when(pl.program_id(0) == 0)
    def _():
        out_ref[...] = jnp.zeros_like(out_ref)
    x = -dec_ref[...]
    sp = jnp.maximum(x, 0.0) + jnp.log1p(jnp.exp(-jnp.abs(x)))
    out_ref[...] += jnp.sum(sp, axis=0, keepdims=True)


def kernel(llr_in, cn_weight, ch_weight, cn_bias, edge_to_vn, edge_to_cn):
    llr_flat = llr_in.reshape(NCHUNK, BC, N).transpose(0, 2, 1).reshape(
        NCHUNK * N, BC)
    vn = edge_to_vn.astype(jnp.int32)
    cn = edge_to_cn.astype(jnp.int32)
    vn_pad = jnp.concatenate([vn, jnp.arange(K, dtype=jnp.int32) % N])
    cn_pad = jnp.concatenate([cn, jnp.full((K,), M, jnp.int32)])
    bins = cn // CN_SB
    cnt = jnp.sum(bins[:, None] == jnp.arange(NSB, dtype=jnp.int32)[None, :],
                  axis=0, dtype=jnp.int32)
    bounds = jnp.concatenate([jnp.zeros((1,), jnp.int32),
                              jnp.cumsum(cnt, dtype=jnp.int32),
                              jnp.full((48 - NSB - 1,), E, jnp.int32)])
    e_lo_i = bounds[:NSB]
    a_lo_i = e_lo_i - e_lo_i % 8
    nb_i = (bounds[1:NSB + 1] - a_lo_i + (K - 1)) // K
    regs = jnp.concatenate([jnp.zeros((1,), jnp.int32),
                            jnp.cumsum(nb_i * K, dtype=jnp.int32),
                            jnp.full((48 - NSB - 1,), 0, jnp.int32)])

    mesh = plsc.VectorSubcoreMesh(core_axis_name="c", subcore_axis_name="s")
    dec, _ = pl.kernel(
        _sc_body,
        out_type=[
            jax.ShapeDtypeStruct((ITERS * NCHUNK * N, BC), jnp.float32),
            jax.ShapeDtypeStruct((NC * CAPMAX, BC), jnp.float32),
        ],
        mesh=mesh,
        compiler_params=pltpu.CompilerParams(use_tc_tiling_on_sc=False),
        scratch_types=[
            pltpu.VMEM((32,), jnp.float32),
            pltpu.VMEM((32,), jnp.float32),
            pltpu.VMEM((32,), jnp.float32),
            pltpu.VMEM((64,), jnp.int32),
            pltpu.VMEM((64,), jnp.int32),
            pltpu.VMEM((K,), jnp.int32),
            pltpu.VMEM((K + 16,), jnp.int32),
            pltpu.VMEM((K, BC), jnp.float32),
            pltpu.VMEM((K, BC), jnp.float32),
            pltpu.VMEM((K, BC), jnp.float32),
            pltpu.VMEM((CN_SB, BC), jnp.float32),
            pltpu.VMEM((CN_SB, BC), jnp.float32),
            pltpu.VMEM((CN_SB, BC), jnp.float32),
            pltpu.VMEM((K,), jnp.int32),
            pltpu.SemaphoreType.DMA,
            pltpu.MemorySpace.VMEM_SHARED((N, BC), jnp.float32),
            pltpu.MemorySpace.VMEM_SHARED((N + 16, BC), jnp.float32),
        ],
    )(llr_flat, vn_pad, cn_pad, bounds, regs,
      jnp.pad(cn_weight.astype(jnp.float32), (0, 16 - ITERS)),
      jnp.pad(ch_weight.astype(jnp.float32), (0, 16 - ITERS)),
      jnp.pad(cn_bias.astype(jnp.float32), (0, 16 - ITERS)))

    BLK = 4096
    nrows = ITERS * NCHUNK * N
    psum = pl.pallas_call(
        _loss_body,
        grid=(nrows // BLK,),
        in_specs=[pl.BlockSpec((BLK, BC), lambda i: (i, 0))],
        out_specs=pl.BlockSpec((1, BC), lambda i: (0, 0)),
        out_shape=jax.ShapeDtypeStruct((1, BC), jnp.float32),
    )(dec)
    return jnp.sum(psum) / (B * N * ITERS)
